# baseline (speedup 1.0000x reference)
import jax
import jax.numpy as jnp
from jax import lax
from jax.experimental import pallas as pl
from jax.experimental.pallas import tpu as pltpu
from jax.experimental.pallas import tpu_sc as plsc

N = 100000
M = 128
H = 128
G = 4096

_S = 1
_NS = N // _S
_BLK = 10000

_CHUNK = 240
_HC = 128
_HC2 = _CHUNK - _HC
_NW = 32
_ROWS_PER_SUB = G // 16


def _gate_body(x1_ref, x2_ref, w1_ref, w2_ref, w3_ref, b1_ref, b2_ref, o_ref):
    x1 = x1_ref[...]
    x2 = x2_ref[...]
    logits = (jnp.dot(x1, w1_ref[...], preferred_element_type=jnp.float32)
              + jnp.dot(x2, w2_ref[...], preferred_element_type=jnp.float32)
              + b1_ref[...])
    att = jax.nn.sigmoid(logits)
    val = jnp.dot(x2, w3_ref[...], preferred_element_type=jnp.float32) + b2_ref[...]
    o_ref[...] = att * val


def _gate_slice(sl, input_rep, final_rep, w1, w2, w3, b1, b2):
    blocks = _NS // _BLK
    off = sl * blocks
    return pl.pallas_call(
        _gate_body,
        grid=(blocks,),
        in_specs=[
            pl.BlockSpec((_BLK, M), lambda i: (i + off, 0)),
            pl.BlockSpec((_BLK, H), lambda i: (i + off, 0)),
            pl.BlockSpec((M, H), lambda i: (0, 0)),
            pl.BlockSpec((H, H), lambda i: (0, 0)),
            pl.BlockSpec((H, H), lambda i: (0, 0)),
            pl.BlockSpec((1, H), lambda i: (0, 0)),
            pl.BlockSpec((1, H), lambda i: (0, 0)),
        ],
        out_specs=pl.BlockSpec((_BLK, H), lambda i: (i, 0)),
        out_shape=jax.ShapeDtypeStruct((_NS, H), jnp.float32),
        name=f"gate_slice{sl}",
    )(input_rep, final_rep, w1, w2, w3, b1, b2)


def _make_segsum_body(row_base):
    nfull = _NS // _CHUNK
    tail = _NS - nfull * _CHUNK
    iters = -(-nfull // _NW)
    full_iters = (nfull - _NW) // _NW + 1
    last_w = nfull - full_iters * _NW

    def body(g_hbm, idx_hbm, out_hbm,
             rows0, rows1, rows2, idxa0, idxa1, idxa2, idxb0, idxb1, idxb2,
             idx_t2, zbuf, acc,
             sg0, sg1, sg2, si0, si1, si2, ss0, ss1, ss2):
        c = lax.axis_index("c")
        s = lax.axis_index("s")
        wid = s * 2 + c
        rows = (rows0, rows1, rows2)
        idxa = (idxa0, idxa1, idxa2)
        idxb = (idxb0, idxb1, idxb2)
        sg = (sg0, sg1, sg2)
        si = (si0, si1, si2)
        ss = (ss0, ss1, ss2)

        def start_loads(t, b):
            base = (wid + t * _NW) * _CHUNK
            pltpu.async_copy(g_hbm.at[pl.ds(base, _CHUNK)], rows[b], sg[b])
            pltpu.async_copy(idx_hbm.at[pl.ds(row_base + base, _HC)], idxa[b], si[b])
            pltpu.async_copy(idx_hbm.at[pl.ds(row_base + base + _HC, _HC2)], idxb[b], si[b])

        def wait_loads(t, b):
            base = (wid + t * _NW) * _CHUNK
            pltpu.make_async_copy(g_hbm.at[pl.ds(base, _CHUNK)], rows[b], sg[b]).wait()
            pltpu.make_async_copy(idx_hbm.at[pl.ds(row_base + base, _HC)], idxa[b], si[b]).wait()
            pltpu.make_async_copy(idx_hbm.at[pl.ds(row_base + base + _HC, _HC2)], idxb[b], si[b]).wait()

        def start_scatter(b):
            pltpu.async_copy(rows[b].at[pl.ds(0, _HC)], acc.at[idxa[b]], ss[b], add=True)
            pltpu.async_copy(rows[b].at[pl.ds(_HC, _HC2)], acc.at[idxb[b]], ss[b], add=True)

        def wait_scatter(b):
            pltpu.make_async_copy(rows[b].at[pl.ds(0, _HC)], acc.at[idxa[b]], ss[b]).wait()
            pltpu.make_async_copy(rows[b].at[pl.ds(_HC, _HC2)], acc.at[idxb[b]], ss[b]).wait()

        nbuf = 3
        start_loads(0, 0)
        if iters > 1:
            start_loads(1, 1)
        zero = jnp.zeros((16,), jnp.float32)
        for i in range(16):
            for j in range(8):
                zbuf[i, pl.ds(j * 16, 16)] = zero
        for k in range(16):
            pltpu.async_copy(zbuf, acc.at[pl.ds(s * _ROWS_PER_SUB + k * 16, 16)], ss0)
        for k in range(16):
            pltpu.make_async_copy(zbuf, acc.at[pl.ds(s * _ROWS_PER_SUB + k * 16, 16)], ss0).wait()
        plsc.subcore_barrier()

        for t in range(iters):
            b = t % nbuf

            def _iter_body(t=t, b=b):
                wait_loads(t, b)
                start_scatter(b)
                tn = t + nbuf - 1
                if tn < iters:
                    if t >= 1:
                        wait_scatter((t - 1) % nbuf)
                    if tn < full_iters:
                        start_loads(tn, tn % nbuf)
                    else:
                        @pl.when(wid < last_w)
                        def _():
                            start_loads(tn, tn % nbuf)

            if t < full_iters:
                _iter_body()
            else:
                pl.when(wid < last_w)(_iter_body)

        for k in range(min(nbuf, iters)):
            ch = iters - 1 - k
            if ch == iters - 1 and iters > full_iters:
                @pl.when(wid < last_w)
                def _(ch=ch):
                    wait_scatter(ch % nbuf)
            else:
                wait_scatter(ch % nbuf)

        if tail:
            @pl.when(wid == _NW - 1)
            def _():
                base = nfull * _CHUNK
                pltpu.sync_copy(g_hbm.at[pl.ds(base, tail)], rows0.at[pl.ds(0, tail)])
                if tail >= _HC:
                    pltpu.sync_copy(idx_hbm.at[pl.ds(row_base + base, _HC)], idxa0)
                    pltpu.sync_copy(rows0.at[pl.ds(0, _HC)], acc.at[idxa0], add=True)
                if tail > _HC:
                    pltpu.sync_copy(idx_hbm.at[pl.ds(row_base + base + _HC, tail - _HC)], idx_t2)
                    pltpu.sync_copy(rows0.at[pl.ds(_HC, tail - _HC)], acc.at[idx_t2], add=True)
                elif tail < _HC:
                    pltpu.sync_copy(idx_hbm.at[pl.ds(row_base + base, tail)], idx_t2)
                    pltpu.sync_copy(rows0.at[pl.ds(0, tail)], acc.at[idx_t2], add=True)

        plsc.subcore_barrier()

        for r in range(2):
            row0 = s * _ROWS_PER_SUB + r * _HC
            pltpu.sync_copy(acc.at[pl.ds(row0, _HC)], rows0.at[pl.ds(0, _HC)])
            pltpu.sync_copy(rows0.at[pl.ds(0, _HC)], out_hbm.at[c, pl.ds(row0, _HC)])

    return body, tail


def _segsum_slice(sl, g, idx):
    row_base = sl * _NS
    body, tail = _make_segsum_body(row_base)
    mesh = plsc.VectorSubcoreMesh(core_axis_name="c", subcore_axis_name="s")
    f = pl.kernel(
        body,
        out_type=jax.ShapeDtypeStruct((2, G, H), jnp.float32),
        mesh=mesh,
        scratch_types=(
            [pltpu.VMEM((_CHUNK, H), jnp.float32)] * 3
            + [pltpu.VMEM((_HC,), jnp.int32)] * 3
            + [pltpu.VMEM((_HC2,), jnp.int32)] * 3
            + [
                pltpu.VMEM((max(tail - _HC if tail > _HC else tail, 8),), jnp.int32),
                pltpu.VMEM((16, H), jnp.float32),
                pltpu.VMEM_SHARED((G, H), jnp.float32),
            ]
            + [pltpu.SemaphoreType.DMA] * 9
        ),
    )
    return f(g, idx)


def _combine_body(*refs):
    p_refs = refs[:-1]
    o_ref = refs[-1]
    acc = p_refs[0][0] + p_refs[0][1]
    for p in p_refs[1:]:
        acc = acc + p[0] + p[1]
    o_ref[...] = acc


def _combine(partials):
    return pl.pallas_call(
        _combine_body,
        grid=(8,),
        in_specs=[pl.BlockSpec((2, G // 8, H), lambda i: (0, i, 0))
                  for _ in partials],
        out_specs=pl.BlockSpec((G // 8, H), lambda i: (i, 0)),
        out_shape=jax.ShapeDtypeStruct((G, H), jnp.float32),
    )(*partials)


@jax.jit
def kernel(input_rep, final_rep, graph_index, W_lin, b_lin, W_last, b_last):
    w1 = W_lin[:, :M].T
    w2 = W_lin[:, M:].T
    w3 = W_last.T
    b1 = b_lin.reshape(1, H)
    b2 = b_last.reshape(1, H)
    idx = graph_index.astype(jnp.int32)
    partials = []
    for sl in range(_S):
        g = _gate_slice(sl, input_rep, final_rep, w1, w2, w3, b1, b2)
        partials.append(_segsum_slice(sl, g, idx))
    return _combine(partials)


# speedup vs baseline: 1.0073x; 1.0073x over previous
import jax
import jax.numpy as jnp
from jax import lax
from jax.experimental import pallas as pl
from jax.experimental.pallas import tpu as pltpu
from jax.experimental.pallas import tpu_sc as plsc

N = 100000
M = 128
H = 128
G = 4096

_S = 1
_NS = N // _S
_BLK = 10000

_CHUNK = 128
_NW = 32
_ROWS_PER_SUB = G // 16


def _gate_body(x1_ref, x2_ref, w1_ref, w2_ref, w3_ref, b1_ref, b2_ref, o_ref):
    x1 = x1_ref[...]
    x2 = x2_ref[...]
    logits = (jnp.dot(x1, w1_ref[...], preferred_element_type=jnp.float32)
              + jnp.dot(x2, w2_ref[...], preferred_element_type=jnp.float32)
              + b1_ref[...])
    att = jax.nn.sigmoid(logits)
    val = jnp.dot(x2, w3_ref[...], preferred_element_type=jnp.float32) + b2_ref[...]
    o_ref[...] = att * val


def _gate_slice(sl, input_rep, final_rep, w1, w2, w3, b1, b2):
    blocks = _NS // _BLK
    off = sl * blocks
    return pl.pallas_call(
        _gate_body,
        grid=(blocks,),
        in_specs=[
            pl.BlockSpec((_BLK, M), lambda i: (i + off, 0)),
            pl.BlockSpec((_BLK, H), lambda i: (i + off, 0)),
            pl.BlockSpec((M, H), lambda i: (0, 0)),
            pl.BlockSpec((H, H), lambda i: (0, 0)),
            pl.BlockSpec((H, H), lambda i: (0, 0)),
            pl.BlockSpec((1, H), lambda i: (0, 0)),
            pl.BlockSpec((1, H), lambda i: (0, 0)),
        ],
        out_specs=pl.BlockSpec((_BLK, H), lambda i: (i, 0)),
        out_shape=jax.ShapeDtypeStruct((_NS, H), jnp.float32),
        name=f"gate_slice{sl}",
    )(input_rep, final_rep, w1, w2, w3, b1, b2)


def _make_segsum_body(row_base):
    nfull = _NS // _CHUNK
    tail = _NS - nfull * _CHUNK
    iters = -(-nfull // _NW)
    full_iters = (nfull - _NW) // _NW + 1
    last_w = nfull - full_iters * _NW

    def body(g_hbm, idx_hbm, out_hbm,
             rows0, rows1, rows2, idx0, idx1, idx2, rows_t, idx_t, zbuf, acc,
             sg0, sg1, sg2, si0, si1, si2, ss0, ss1, ss2):
        c = lax.axis_index("c")
        s = lax.axis_index("s")
        wid = s * 2 + c
        rows = (rows0, rows1, rows2)
        idxb = (idx0, idx1, idx2)
        sg = (sg0, sg1, sg2)
        si = (si0, si1, si2)
        ss = (ss0, ss1, ss2)

        def start_loads(t, b):
            base = (wid + t * _NW) * _CHUNK
            pltpu.async_copy(g_hbm.at[pl.ds(base, _CHUNK)], rows[b], sg[b])
            pltpu.async_copy(idx_hbm.at[pl.ds(row_base + base, _CHUNK)], idxb[b], si[b])

        def wait_loads(t, b):
            base = (wid + t * _NW) * _CHUNK
            pltpu.make_async_copy(g_hbm.at[pl.ds(base, _CHUNK)], rows[b], sg[b]).wait()
            pltpu.make_async_copy(idx_hbm.at[pl.ds(row_base + base, _CHUNK)], idxb[b], si[b]).wait()

        def start_scatter(b):
            pltpu.async_copy(rows[b], acc.at[idxb[b]], ss[b], add=True)

        def wait_scatter(b):
            pltpu.make_async_copy(rows[b], acc.at[idxb[b]], ss[b]).wait()

        nbuf = 3
        start_loads(0, 0)
        if iters > 1:
            start_loads(1, 1)
        zero = jnp.zeros((16,), jnp.float32)
        for i in range(16):
            for j in range(8):
                zbuf[i, pl.ds(j * 16, 16)] = zero
        for k in range(16):
            pltpu.async_copy(zbuf, acc.at[pl.ds(s * _ROWS_PER_SUB + k * 16, 16)], ss0)
        for k in range(16):
            pltpu.make_async_copy(zbuf, acc.at[pl.ds(s * _ROWS_PER_SUB + k * 16, 16)], ss0).wait()
        plsc.subcore_barrier()

        for t in range(iters):
            b = t % nbuf

            def _iter_body(t=t, b=b):
                wait_loads(t, b)
                start_scatter(b)
                tn = t + nbuf - 1
                if tn < iters:
                    if t >= 1:
                        wait_scatter((t - 1) % nbuf)
                    if tn < full_iters:
                        start_loads(tn, tn % nbuf)
                    else:
                        @pl.when(wid < last_w)
                        def _():
                            start_loads(tn, tn % nbuf)

            if t < full_iters:
                _iter_body()
            else:
                pl.when(wid < last_w)(_iter_body)

        for k in range(min(nbuf, iters)):
            ch = iters - 1 - k
            if ch == iters - 1 and iters > full_iters:
                @pl.when(wid < last_w)
                def _(ch=ch):
                    wait_scatter(ch % nbuf)
            else:
                wait_scatter(ch % nbuf)

        if tail:
            @pl.when(wid == _NW - 1)
            def _():
                base = nfull * _CHUNK
                pltpu.sync_copy(g_hbm.at[pl.ds(base, tail)], rows_t)
                pltpu.sync_copy(idx_hbm.at[pl.ds(row_base + base, tail)], idx_t)
                pltpu.sync_copy(rows_t, acc.at[idx_t], add=True)

        plsc.subcore_barrier()

        nw_rounds = _ROWS_PER_SUB // _CHUNK
        for r in range(nw_rounds):
            row0 = s * _ROWS_PER_SUB + r * _CHUNK
            pltpu.async_copy(acc.at[pl.ds(row0, _CHUNK)], rows[r % 3], sg[r % 3])
        for r in range(nw_rounds):
            row0 = s * _ROWS_PER_SUB + r * _CHUNK
            pltpu.make_async_copy(acc.at[pl.ds(row0, _CHUNK)], rows[r % 3], sg[r % 3]).wait()
            pltpu.async_copy(rows[r % 3], out_hbm.at[c, pl.ds(row0, _CHUNK)], si[r % 3])
        for r in range(nw_rounds):
            row0 = s * _ROWS_PER_SUB + r * _CHUNK
            pltpu.make_async_copy(rows[r % 3], out_hbm.at[c, pl.ds(row0, _CHUNK)], si[r % 3]).wait()

    return body, tail


def _segsum_slice(sl, g, idx):
    row_base = sl * _NS
    body, tail = _make_segsum_body(row_base)
    mesh = plsc.VectorSubcoreMesh(core_axis_name="c", subcore_axis_name="s")
    f = pl.kernel(
        body,
        out_type=jax.ShapeDtypeStruct((2, G, H), jnp.float32),
        mesh=mesh,
        scratch_types=(
            [pltpu.VMEM((_CHUNK, H), jnp.float32)] * 3
            + [pltpu.VMEM((_CHUNK,), jnp.int32)] * 3
            + [
                pltpu.VMEM((max(tail, 8), H), jnp.float32),
                pltpu.VMEM((max(tail, 8),), jnp.int32),
                pltpu.VMEM((16, H), jnp.float32),
                pltpu.VMEM_SHARED((G, H), jnp.float32),
            ]
            + [pltpu.SemaphoreType.DMA] * 9
        ),
    )
    return f(g, idx)


def _combine_body(*refs):
    p_refs = refs[:-1]
    o_ref = refs[-1]
    acc = p_refs[0][0] + p_refs[0][1]
    for p in p_refs[1:]:
        acc = acc + p[0] + p[1]
    o_ref[...] = acc


def _combine(partials):
    return pl.pallas_call(
        _combine_body,
        grid=(8,),
        in_specs=[pl.BlockSpec((2, G // 8, H), lambda i: (0, i, 0))
                  for _ in partials],
        out_specs=pl.BlockSpec((G // 8, H), lambda i: (i, 0)),
        out_shape=jax.ShapeDtypeStruct((G, H), jnp.float32),
    )(*partials)


@jax.jit
def kernel(input_rep, final_rep, graph_index, W_lin, b_lin, W_last, b_last):
    w1 = W_lin[:, :M].T
    w2 = W_lin[:, M:].T
    w3 = W_last.T
    b1 = b_lin.reshape(1, H)
    b2 = b_last.reshape(1, H)
    idx = graph_index.astype(jnp.int32)
    partials = []
    for sl in range(_S):
        g = _gate_slice(sl, input_rep, final_rep, w1, w2, w3, b1, b2)
        partials.append(_segsum_slice(sl, g, idx))
    return _combine(partials)
